# baseline (speedup 1.0000x reference)
import functools

import jax
import jax.numpy as jnp
import numpy as np
from jax import lax
from jax.experimental import pallas as pl
from jax.experimental.pallas import tpu as pltpu
from jax.experimental.pallas import tpu_sc as plsc

_DN = (((1,), (0,)), ((), ()))
_BLK = 2000

_info = plsc.get_sparse_core_info()
_NC, _NS = _info.num_cores, _info.num_subcores
_NW = _NC * _NS

_E = 320000
_K = _E // 2
_NT = 16
_CHUNK = _E // _NT
_SUB = _CHUNK // 16
_R = 256
_NPASS = 4
_SROWS = _CHUNK // 128 + 1
_PAD = 4096
_SIGN = np.int32(-2147483648)
_BUFSWAP = False

_OCHUNK = _K // _NT
_OROWS = _OCHUNK // 128 + 1


def _dot_body(c_ref, a_ref, b_ref, w_ref, o_ref):
    er = jnp.concatenate([a_ref[...], b_ref[...]], axis=1).astype(jnp.bfloat16)
    acc = lax.dot_general(er, w_ref[...], _DN, preferred_element_type=jnp.float32)
    o_ref[...] = acc + c_ref[0]


def _score(er0, er1, W, c):
    E = er0.shape[0]
    n_blk = E // _BLK
    return pl.pallas_call(
        _dot_body,
        grid=(n_blk,),
        in_specs=[
            pl.BlockSpec(memory_space=pltpu.SMEM),
            pl.BlockSpec((_BLK, 128), lambda i: (i, 0)),
            pl.BlockSpec((_BLK, 128), lambda i: (i, 0)),
            pl.BlockSpec((256, 1), lambda i: (0, 0)),
        ],
        out_specs=pl.BlockSpec((_BLK, 1), lambda i: (i, 0)),
        out_shape=jax.ShapeDtypeStruct((E, 1), jnp.float32),
    )(c, er0, er1, W)


def _gather_rows(xb32, row, col):
    N, D = xb32.shape
    E = row.shape[0]
    per_w = E // _NW
    CH = 200
    n_ch = per_w // CH
    mesh = plsc.VectorSubcoreMesh(core_axis_name="c", subcore_axis_name="s")

    @functools.partial(
        pl.kernel,
        mesh=mesh,
        out_type=[
            jax.ShapeDtypeStruct((E, D), jnp.float32),
            jax.ShapeDtypeStruct((E, D), jnp.float32),
        ],
        scratch_types=[
            pltpu.VMEM((CH,), jnp.int32),
            pltpu.VMEM((CH, 128), jnp.float32),
            pltpu.SemaphoreType.DMA,
        ],
    )
    def k(x_hbm, row_hbm, col_hbm, er0_hbm, er1_hbm, idx_v, rows_v, sem):
        wid = lax.axis_index("s") * _NC + lax.axis_index("c")
        base = wid * per_w

        def body(j, _):
            off = base + j * CH
            pltpu.sync_copy(row_hbm.at[pl.ds(off, CH)], idx_v)
            pltpu.async_copy(x_hbm.at[idx_v], rows_v, sem).wait()
            pltpu.sync_copy(rows_v, er0_hbm.at[pl.ds(off, CH), :])
            pltpu.sync_copy(col_hbm.at[pl.ds(off, CH)], idx_v)
            pltpu.async_copy(x_hbm.at[idx_v], rows_v, sem).wait()
            pltpu.sync_copy(rows_v, er1_hbm.at[pl.ds(off, CH), :])
            return 0

        lax.fori_loop(0, n_ch, body, 0)

    return k(xb32, row, col)


def _sort_topk(pred_bits):
    mesh = plsc.VectorSubcoreMesh(
        core_axis_name="c", subcore_axis_name="s", num_cores=1)

    @functools.partial(
        pl.kernel,
        mesh=mesh,
        out_type=[
            jax.ShapeDtypeStruct((_K,), jnp.int32),
            jax.ShapeDtypeStruct((_K,), jnp.int32),
            jax.ShapeDtypeStruct((_E + _PAD,), jnp.int32),
            jax.ShapeDtypeStruct((_E + _PAD,), jnp.int32),
        ],
        scratch_types=[
            pltpu.VMEM_SHARED((_E + _PAD,), jnp.int32),
            pltpu.VMEM_SHARED((_E + _PAD,), jnp.int32),
            pltpu.VMEM_SHARED((_R * _NT,), jnp.int32),
            pltpu.VMEM((_SROWS * 128,), jnp.int32),
            pltpu.VMEM((_SROWS * 128,), jnp.int32),
            pltpu.VMEM((_SROWS, 128), jnp.int32),
            pltpu.VMEM((_R * 16,), jnp.int32),
            pltpu.VMEM((_R * 16,), jnp.int32),
            pltpu.VMEM((_R * _NT,), jnp.int32),
            pltpu.VMEM((_R,), jnp.int32),
            pltpu.VMEM((16,), jnp.int32),
            pltpu.SemaphoreType.DMA,
        ],
        compiler_params=pltpu.CompilerParams(needs_layout_passes=False),
    )
    def k(pred_hbm, key_out, idx_out, skA, svA,
          skB, svB, hist_sh, ck, cv, dests, lhist, counters,
          grid_v, totals_v, spill_v, sem):
        cid = lax.axis_index("c")
        sid = lax.axis_index("s")
        t = sid
        lane = jnp.arange(16, dtype=jnp.int32)
        cbase = t * _CHUNK

        @pl.when(cid == 0)
        def _sort():
            def fill_iota(jj, _):
                cv[pl.ds(jj * 16, 16)] = cbase + jj * 16 + lane
                return 0
            lax.fori_loop(0, _CHUNK // 16, fill_iota, 0)

            for p in range(_NPASS):
                src_k, src_v = skA, svA
                dst_k, dst_v = skB, svB
                shift = 8 * p
                if p == 0:
                    pltpu.sync_copy(pred_hbm.at[pl.ds(cbase, _CHUNK)],
                                    ck.at[pl.ds(0, _CHUNK)])

                    def xform(jj, _):
                        bb = ck[pl.ds(jj * 16, 16)]
                        ck[pl.ds(jj * 16, 16)] = jnp.where(
                            bb < 0, bb,
                            jnp.bitwise_xor(jnp.bitwise_not(bb), _SIGN))
                        return 0
                    lax.fori_loop(0, _CHUNK // 16, xform, 0)
                else:
                    pltpu.sync_copy(src_k.at[pl.ds(cbase, _CHUNK)],
                                    ck.at[pl.ds(0, _CHUNK)])
                    pltpu.sync_copy(src_v.at[pl.ds(cbase, _CHUNK)],
                                    cv.at[pl.ds(0, _CHUNK)])

                def zero_h(i, _):
                    lhist[pl.ds(i * 16, 16)] = jnp.zeros((16,), jnp.int32)
                    return 0
                lax.fori_loop(0, _R, zero_h, 0)

                def hist_body(j, _):
                    kk = plsc.load_gather(ck, [lane * _SUB + j])
                    d = lax.shift_right_logical(kk, shift) & 255
                    cidx = lane * _R + d
                    old = plsc.load_gather(lhist, [cidx])
                    plsc.store_scatter(lhist, [cidx], old + 1)
                    return 0
                lax.fori_loop(0, _SUB, hist_body, 0)

                def tot_body(q, _):
                    d = q * 16 + lane
                    acc = jnp.zeros((16,), jnp.int32)
                    for l in range(16):
                        acc = acc + plsc.load_gather(lhist, [l * _R + d])
                    totals_v[pl.ds(q * 16, 16)] = acc
                    return 0
                lax.fori_loop(0, _R // 16, tot_body, 0)
                pltpu.sync_copy(totals_v, hist_sh.at[pl.ds(t * _R, _R)])
                plsc.subcore_barrier()

                pltpu.sync_copy(hist_sh, grid_v)

                def scan_body(d, run):
                    v = plsc.load_gather(grid_v, [lane * _R + d])
                    cs = plsc.cumsum(v)
                    ex = cs - v
                    spill_v[...] = run + ex
                    my_base = plsc.load_gather(
                        spill_v, [jnp.full((16,), t, jnp.int32)])
                    lh = plsc.load_gather(lhist, [lane * _R + d])
                    lcs = plsc.cumsum(lh)
                    lex = lcs - lh
                    plsc.store_scatter(counters, [lane * _R + d], my_base + lex)
                    spill_v[...] = cs
                    tot = plsc.load_gather(
                        spill_v, [jnp.full((16,), 15, jnp.int32)])
                    return run + tot

                lax.fori_loop(0, _R, scan_body, jnp.zeros((16,), jnp.int32))

                def rank_body(j, _):
                    pos = lane * _SUB + j
                    kk = plsc.load_gather(ck, [pos])
                    d = lax.shift_right_logical(kk, shift) & 255
                    cidx = lane * _R + d
                    dest = plsc.load_gather(counters, [cidx])
                    plsc.store_scatter(counters, [cidx], dest + 1)
                    plsc.store_scatter(dests, [pos // 128, pos % 128], dest)
                    return 0
                lax.fori_loop(0, _SUB, rank_body, 0)

                def pad_body(j, _):
                    pos = _CHUNK + j * 16 + lane
                    plsc.store_scatter(dests, [pos // 128, pos % 128],
                                       _E + t * 128 + j * 16 + lane)
                    return 0
                lax.fori_loop(0, (_SROWS * 128 - _CHUNK) // 16, pad_body, 0)

                def scat_body(f, _):
                    pltpu.async_copy(ck.at[pl.ds(f * 128, 128)],
                                     dst_k.at[dests.at[f]], sem)
                    pltpu.make_async_copy(ck.at[pl.ds(f * 128, 128)],
                                          dst_k.at[dests.at[f]], sem).wait()
                    pltpu.async_copy(cv.at[pl.ds(f * 128, 128)],
                                     dst_v.at[dests.at[f]], sem)
                    pltpu.make_async_copy(cv.at[pl.ds(f * 128, 128)],
                                          dst_v.at[dests.at[f]], sem).wait()
                    return 0
                lax.fori_loop(0, _SROWS, scat_body, 0)
                plsc.subcore_barrier()

                if p < _NPASS - 1:
                    pltpu.sync_copy(skB.at[pl.ds(cbase, _CHUNK)],
                                    ck.at[pl.ds(0, _CHUNK)])
                    pltpu.sync_copy(ck.at[pl.ds(0, _CHUNK)],
                                    skA.at[pl.ds(cbase, _CHUNK)])
                    pltpu.sync_copy(svB.at[pl.ds(cbase, _CHUNK)],
                                    cv.at[pl.ds(0, _CHUNK)])
                    pltpu.sync_copy(cv.at[pl.ds(0, _CHUNK)],
                                    svA.at[pl.ds(cbase, _CHUNK)])
                    plsc.subcore_barrier()

            fin_k, fin_v = skB, svB
            obase = t * _OCHUNK
            pltpu.sync_copy(fin_k.at[pl.ds(obase, _OCHUNK)],
                            ck.at[pl.ds(0, _OCHUNK)])
            pltpu.sync_copy(fin_v.at[pl.ds(obase, _OCHUNK)],
                            cv.at[pl.ds(0, _OCHUNK)])
            pltpu.sync_copy(ck.at[pl.ds(0, _OCHUNK)],
                            key_out.at[pl.ds(obase, _OCHUNK)])
            pltpu.sync_copy(cv.at[pl.ds(0, _OCHUNK)],
                            idx_out.at[pl.ds(obase, _OCHUNK)])

    return k(pred_bits)


_FCH = _K // _NW
_FROWS = _FCH // 128 + 1


def _finalize(sk, sv, row, col):
    mesh = plsc.VectorSubcoreMesh(core_axis_name="c", subcore_axis_name="s")

    @functools.partial(
        pl.kernel,
        mesh=mesh,
        out_type=[
            jax.ShapeDtypeStruct((_K,), jnp.float32),
            jax.ShapeDtypeStruct((_K,), jnp.int32),
            jax.ShapeDtypeStruct((_K,), jnp.int32),
        ],
        scratch_types=[
            pltpu.VMEM((_FCH,), jnp.int32),
            pltpu.VMEM((_FCH,), jnp.float32),
            pltpu.VMEM((_FROWS, 128), jnp.int32),
            pltpu.VMEM((_FROWS * 128,), jnp.int32),
            pltpu.SemaphoreType.DMA,
        ],
        compiler_params=pltpu.CompilerParams(needs_layout_passes=False),
    )
    def k(sk_hbm, sv_hbm, row_hbm, col_hbm, vals_out, eir_out, eic_out,
          buf, fvals, oidx, obuf, sem):
        wid = lax.axis_index("s") * _NC + lax.axis_index("c")
        base = wid * _FCH
        lane = jnp.arange(16, dtype=jnp.int32)

        pltpu.sync_copy(sk_hbm.at[pl.ds(base, _FCH)], buf)

        def inv_one(off):
            k2 = buf[pl.ds(off, 16)]
            bb = jnp.where(k2 < 0, k2,
                           jnp.bitwise_xor(jnp.bitwise_not(k2), _SIGN))
            fvals[pl.ds(off, 16)] = plsc.bitcast(bb, jnp.float32)

        def inv_body(jj, _):
            inv_one(jj * 16)
            return 0
        lax.fori_loop(0, _FCH // 16, inv_body, 0)
        if _FCH % 16:
            inv_one(_FCH - 16)
        pltpu.sync_copy(fvals, vals_out.at[pl.ds(base, _FCH)])

        pltpu.sync_copy(sv_hbm.at[pl.ds(base, _FCH)], buf)

        def oidx_fill(jj, _):
            pos = jj * 16 + lane
            v = jnp.where(pos < _FCH,
                          plsc.load_gather(buf, [jnp.minimum(pos, _FCH - 1)]),
                          0)
            plsc.store_scatter(oidx, [pos // 128, pos % 128], v)
            return 0
        lax.fori_loop(0, (_FROWS * 128) // 16, oidx_fill, 0)

        def gat_r(f, _):
            pltpu.async_copy(row_hbm.at[oidx.at[f]],
                             obuf.at[pl.ds(f * 128, 128)], sem)
            pltpu.make_async_copy(row_hbm.at[oidx.at[f]],
                                  obuf.at[pl.ds(f * 128, 128)], sem).wait()
            return 0
        lax.fori_loop(0, _FROWS, gat_r, 0)
        pltpu.sync_copy(obuf.at[pl.ds(0, _FCH)],
                        eir_out.at[pl.ds(base, _FCH)])

        def gat_c(f, _):
            pltpu.async_copy(col_hbm.at[oidx.at[f]],
                             obuf.at[pl.ds(f * 128, 128)], sem)
            pltpu.make_async_copy(col_hbm.at[oidx.at[f]],
                                  obuf.at[pl.ds(f * 128, 128)], sem).wait()
            return 0
        lax.fori_loop(0, _FROWS, gat_c, 0)
        pltpu.sync_copy(obuf.at[pl.ds(0, _FCH)],
                        eic_out.at[pl.ds(base, _FCH)])

    return k(sk, sv, row, col)


def kernel(x, edge_index, W, b, k):
    row = edge_index[0]
    col = edge_index[1]
    xb32 = x.astype(jnp.bfloat16).astype(jnp.float32)
    er0, er1 = _gather_rows(xb32, row, col)
    k_static = edge_index.shape[1] // 2
    k_residual = (jnp.asarray(k) - k_static).astype(jnp.float32)
    c = (b[0] + k_residual).reshape(1)
    pred = _score(er0, er1, W, c).reshape(-1)
    pred_bits = lax.bitcast_convert_type(pred, jnp.int32)
    sk, sv, _hk, _hv = _sort_topk(pred_bits)
    causal_vals, eir, eic = _finalize(sk, sv, row, col)
    causal_idx = sv
    causal_edge_index = jnp.stack([eir, eic])
    return (causal_vals, causal_idx, causal_edge_index)


# speedup vs baseline: 1.0875x; 1.0875x over previous
import functools

import jax
import jax.numpy as jnp
import numpy as np
from jax import lax
from jax.experimental import pallas as pl
from jax.experimental.pallas import tpu as pltpu
from jax.experimental.pallas import tpu_sc as plsc

_DN = (((1,), (0,)), ((), ()))
_BLK = 2000

_info = plsc.get_sparse_core_info()
_NC, _NS = _info.num_cores, _info.num_subcores
_NW = _NC * _NS

_E = 320000
_K = _E // 2
_NT = 16
_CHUNK = _E // _NT
_SUB = _CHUNK // 16
_R = 256
_NPASS = 4
_SROWS = _CHUNK // 128 + 1
_PAD = 4096
_SIGN = np.int32(-2147483648)
_BUFSWAP = False

_OCHUNK = _K // _NT
_OROWS = _OCHUNK // 128 + 1


def _dot_body(c_ref, a_ref, b_ref, w_ref, o_ref):
    er = jnp.concatenate([a_ref[...], b_ref[...]], axis=1).astype(jnp.bfloat16)
    acc = lax.dot_general(er, w_ref[...], _DN, preferred_element_type=jnp.float32)
    o_ref[...] = acc + c_ref[0]


def _score(er0, er1, W, c):
    E = er0.shape[0]
    n_blk = E // _BLK
    return pl.pallas_call(
        _dot_body,
        grid=(n_blk,),
        in_specs=[
            pl.BlockSpec(memory_space=pltpu.SMEM),
            pl.BlockSpec((_BLK, 128), lambda i: (i, 0)),
            pl.BlockSpec((_BLK, 128), lambda i: (i, 0)),
            pl.BlockSpec((256, 1), lambda i: (0, 0)),
        ],
        out_specs=pl.BlockSpec((_BLK, 1), lambda i: (i, 0)),
        out_shape=jax.ShapeDtypeStruct((E, 1), jnp.float32),
    )(c, er0, er1, W)


def _gather_rows(xb32, row, col):
    N, D = xb32.shape
    E = row.shape[0]
    per_w = E // _NW
    CH = 400
    n_ch = per_w // CH
    mesh = plsc.VectorSubcoreMesh(core_axis_name="c", subcore_axis_name="s")

    @functools.partial(
        pl.kernel,
        mesh=mesh,
        out_type=[
            jax.ShapeDtypeStruct((E, D), jnp.float32),
            jax.ShapeDtypeStruct((E, D), jnp.float32),
        ],
        scratch_types=[
            pltpu.VMEM((CH,), jnp.int32),
            pltpu.VMEM((CH,), jnp.int32),
            pltpu.VMEM((CH, 128), jnp.float32),
            pltpu.VMEM((CH, 128), jnp.float32),
            pltpu.SemaphoreType.DMA,
            pltpu.SemaphoreType.DMA,
        ],
    )
    def k(x_hbm, row_hbm, col_hbm, er0_hbm, er1_hbm,
          idx_r, idx_c, rows_r, rows_c, sem1, sem2):
        wid = lax.axis_index("s") * _NC + lax.axis_index("c")
        base = wid * per_w

        def body(j, _):
            off = base + j * CH
            pltpu.sync_copy(row_hbm.at[pl.ds(off, CH)], idx_r)
            pltpu.sync_copy(col_hbm.at[pl.ds(off, CH)], idx_c)
            a1 = pltpu.async_copy(x_hbm.at[idx_r], rows_r, sem1)
            a2 = pltpu.async_copy(x_hbm.at[idx_c], rows_c, sem2)
            a1.wait()
            pltpu.sync_copy(rows_r, er0_hbm.at[pl.ds(off, CH), :])
            a2.wait()
            pltpu.sync_copy(rows_c, er1_hbm.at[pl.ds(off, CH), :])
            return 0

        lax.fori_loop(0, n_ch, body, 0)

    return k(xb32, row, col)


def _sort_topk(pred_bits):
    mesh = plsc.VectorSubcoreMesh(
        core_axis_name="c", subcore_axis_name="s", num_cores=1)

    @functools.partial(
        pl.kernel,
        mesh=mesh,
        out_type=[
            jax.ShapeDtypeStruct((_K,), jnp.int32),
            jax.ShapeDtypeStruct((_K,), jnp.int32),
            jax.ShapeDtypeStruct((_E + _PAD,), jnp.int32),
            jax.ShapeDtypeStruct((_E + _PAD,), jnp.int32),
        ],
        scratch_types=[
            pltpu.VMEM_SHARED((_E + _PAD,), jnp.int32),
            pltpu.VMEM_SHARED((_E + _PAD,), jnp.int32),
            pltpu.VMEM_SHARED((_R * _NT,), jnp.int32),
            pltpu.VMEM((_SROWS * 128,), jnp.int32),
            pltpu.VMEM((_SROWS * 128,), jnp.int32),
            pltpu.VMEM((_SROWS, 128), jnp.int32),
            pltpu.VMEM((_R * 16,), jnp.int32),
            pltpu.VMEM((_R * 16,), jnp.int32),
            pltpu.VMEM((_R * _NT,), jnp.int32),
            pltpu.VMEM((_R,), jnp.int32),
            pltpu.VMEM((16,), jnp.int32),
            pltpu.SemaphoreType.DMA,
        ],
        compiler_params=pltpu.CompilerParams(needs_layout_passes=False),
    )
    def k(pred_hbm, key_out, idx_out, skA, svA,
          skB, svB, hist_sh, ck, cv, dests, lhist, counters,
          grid_v, totals_v, spill_v, sem):
        cid = lax.axis_index("c")
        sid = lax.axis_index("s")
        t = sid
        lane = jnp.arange(16, dtype=jnp.int32)
        cbase = t * _CHUNK

        @pl.when(cid == 0)
        def _sort():
            def fill_iota(jj, _):
                cv[pl.ds(jj * 16, 16)] = cbase + jj * 16 + lane
                return 0
            lax.fori_loop(0, _CHUNK // 16, fill_iota, 0)

            for p in range(_NPASS):
                src_k, src_v = skA, svA
                dst_k, dst_v = skB, svB
                shift = 8 * p
                if p == 0:
                    pltpu.sync_copy(pred_hbm.at[pl.ds(cbase, _CHUNK)],
                                    ck.at[pl.ds(0, _CHUNK)])

                    def xform(jj, _):
                        bb = ck[pl.ds(jj * 16, 16)]
                        ck[pl.ds(jj * 16, 16)] = jnp.where(
                            bb < 0, bb,
                            jnp.bitwise_xor(jnp.bitwise_not(bb), _SIGN))
                        return 0
                    lax.fori_loop(0, _CHUNK // 16, xform, 0)
                else:
                    pltpu.sync_copy(src_k.at[pl.ds(cbase, _CHUNK)],
                                    ck.at[pl.ds(0, _CHUNK)])
                    pltpu.sync_copy(src_v.at[pl.ds(cbase, _CHUNK)],
                                    cv.at[pl.ds(0, _CHUNK)])

                def zero_h(i, _):
                    lhist[pl.ds(i * 16, 16)] = jnp.zeros((16,), jnp.int32)
                    return 0
                lax.fori_loop(0, _R, zero_h, 0)

                def hist_body(j, _):
                    kk = plsc.load_gather(ck, [lane * _SUB + j])
                    d = lax.shift_right_logical(kk, shift) & 255
                    cidx = lane * _R + d
                    old = plsc.load_gather(lhist, [cidx])
                    plsc.store_scatter(lhist, [cidx], old + 1)
                    return 0
                lax.fori_loop(0, _SUB, hist_body, 0)

                def tot_body(q, _):
                    d = q * 16 + lane
                    acc = jnp.zeros((16,), jnp.int32)
                    for l in range(16):
                        acc = acc + plsc.load_gather(lhist, [l * _R + d])
                    totals_v[pl.ds(q * 16, 16)] = acc
                    return 0
                lax.fori_loop(0, _R // 16, tot_body, 0)
                pltpu.sync_copy(totals_v, hist_sh.at[pl.ds(t * _R, _R)])
                plsc.subcore_barrier()

                pltpu.sync_copy(hist_sh, grid_v)

                def scan_body(d, run):
                    v = plsc.load_gather(grid_v, [lane * _R + d])
                    cs = plsc.cumsum(v)
                    ex = cs - v
                    spill_v[...] = run + ex
                    my_base = plsc.load_gather(
                        spill_v, [jnp.full((16,), t, jnp.int32)])
                    lh = plsc.load_gather(lhist, [lane * _R + d])
                    lcs = plsc.cumsum(lh)
                    lex = lcs - lh
                    plsc.store_scatter(counters, [lane * _R + d], my_base + lex)
                    spill_v[...] = cs
                    tot = plsc.load_gather(
                        spill_v, [jnp.full((16,), 15, jnp.int32)])
                    return run + tot

                lax.fori_loop(0, _R, scan_body, jnp.zeros((16,), jnp.int32))

                def rank_body(j, _):
                    pos = lane * _SUB + j
                    kk = plsc.load_gather(ck, [pos])
                    d = lax.shift_right_logical(kk, shift) & 255
                    cidx = lane * _R + d
                    dest = plsc.load_gather(counters, [cidx])
                    plsc.store_scatter(counters, [cidx], dest + 1)
                    plsc.store_scatter(dests, [pos // 128, pos % 128], dest)
                    return 0
                lax.fori_loop(0, _SUB, rank_body, 0)

                def pad_body(j, _):
                    pos = _CHUNK + j * 16 + lane
                    plsc.store_scatter(dests, [pos // 128, pos % 128],
                                       _E + t * 128 + j * 16 + lane)
                    return 0
                lax.fori_loop(0, (_SROWS * 128 - _CHUNK) // 16, pad_body, 0)

                def scat_body(f, _):
                    pltpu.async_copy(ck.at[pl.ds(f * 128, 128)],
                                     dst_k.at[dests.at[f]], sem)
                    pltpu.make_async_copy(ck.at[pl.ds(f * 128, 128)],
                                          dst_k.at[dests.at[f]], sem).wait()
                    pltpu.async_copy(cv.at[pl.ds(f * 128, 128)],
                                     dst_v.at[dests.at[f]], sem)
                    pltpu.make_async_copy(cv.at[pl.ds(f * 128, 128)],
                                          dst_v.at[dests.at[f]], sem).wait()
                    return 0
                lax.fori_loop(0, _SROWS, scat_body, 0)
                plsc.subcore_barrier()

                if p < _NPASS - 1:
                    pltpu.sync_copy(skB.at[pl.ds(cbase, _CHUNK)],
                                    ck.at[pl.ds(0, _CHUNK)])
                    pltpu.sync_copy(ck.at[pl.ds(0, _CHUNK)],
                                    skA.at[pl.ds(cbase, _CHUNK)])
                    pltpu.sync_copy(svB.at[pl.ds(cbase, _CHUNK)],
                                    cv.at[pl.ds(0, _CHUNK)])
                    pltpu.sync_copy(cv.at[pl.ds(0, _CHUNK)],
                                    svA.at[pl.ds(cbase, _CHUNK)])
                    plsc.subcore_barrier()

            fin_k, fin_v = skB, svB
            obase = t * _OCHUNK
            pltpu.sync_copy(fin_k.at[pl.ds(obase, _OCHUNK)],
                            ck.at[pl.ds(0, _OCHUNK)])
            pltpu.sync_copy(fin_v.at[pl.ds(obase, _OCHUNK)],
                            cv.at[pl.ds(0, _OCHUNK)])
            pltpu.sync_copy(ck.at[pl.ds(0, _OCHUNK)],
                            key_out.at[pl.ds(obase, _OCHUNK)])
            pltpu.sync_copy(cv.at[pl.ds(0, _OCHUNK)],
                            idx_out.at[pl.ds(obase, _OCHUNK)])

    return k(pred_bits)


_FCH = _K // _NW
_FROWS = _FCH // 128 + 1


def _finalize(sk, sv, row, col):
    mesh = plsc.VectorSubcoreMesh(core_axis_name="c", subcore_axis_name="s")

    @functools.partial(
        pl.kernel,
        mesh=mesh,
        out_type=[
            jax.ShapeDtypeStruct((_K,), jnp.float32),
            jax.ShapeDtypeStruct((_K,), jnp.int32),
            jax.ShapeDtypeStruct((_K,), jnp.int32),
        ],
        scratch_types=[
            pltpu.VMEM((_FCH,), jnp.int32),
            pltpu.VMEM((_FCH,), jnp.float32),
            pltpu.VMEM((_FROWS, 128), jnp.int32),
            pltpu.VMEM((_FROWS * 128,), jnp.int32),
            pltpu.SemaphoreType.DMA,
        ],
        compiler_params=pltpu.CompilerParams(needs_layout_passes=False),
    )
    def k(sk_hbm, sv_hbm, row_hbm, col_hbm, vals_out, eir_out, eic_out,
          buf, fvals, oidx, obuf, sem):
        wid = lax.axis_index("s") * _NC + lax.axis_index("c")
        base = wid * _FCH
        lane = jnp.arange(16, dtype=jnp.int32)

        pltpu.sync_copy(sk_hbm.at[pl.ds(base, _FCH)], buf)

        def inv_one(off):
            k2 = buf[pl.ds(off, 16)]
            bb = jnp.where(k2 < 0, k2,
                           jnp.bitwise_xor(jnp.bitwise_not(k2), _SIGN))
            fvals[pl.ds(off, 16)] = plsc.bitcast(bb, jnp.float32)

        def inv_body(jj, _):
            inv_one(jj * 16)
            return 0
        lax.fori_loop(0, _FCH // 16, inv_body, 0)
        if _FCH % 16:
            inv_one(_FCH - 16)
        pltpu.sync_copy(fvals, vals_out.at[pl.ds(base, _FCH)])

        pltpu.sync_copy(sv_hbm.at[pl.ds(base, _FCH)], buf)

        def oidx_fill(jj, _):
            pos = jj * 16 + lane
            v = jnp.where(pos < _FCH,
                          plsc.load_gather(buf, [jnp.minimum(pos, _FCH - 1)]),
                          0)
            plsc.store_scatter(oidx, [pos // 128, pos % 128], v)
            return 0
        lax.fori_loop(0, (_FROWS * 128) // 16, oidx_fill, 0)

        def gat_r(f, _):
            pltpu.async_copy(row_hbm.at[oidx.at[f]],
                             obuf.at[pl.ds(f * 128, 128)], sem)
            pltpu.make_async_copy(row_hbm.at[oidx.at[f]],
                                  obuf.at[pl.ds(f * 128, 128)], sem).wait()
            return 0
        lax.fori_loop(0, _FROWS, gat_r, 0)
        pltpu.sync_copy(obuf.at[pl.ds(0, _FCH)],
                        eir_out.at[pl.ds(base, _FCH)])

        def gat_c(f, _):
            pltpu.async_copy(col_hbm.at[oidx.at[f]],
                             obuf.at[pl.ds(f * 128, 128)], sem)
            pltpu.make_async_copy(col_hbm.at[oidx.at[f]],
                                  obuf.at[pl.ds(f * 128, 128)], sem).wait()
            return 0
        lax.fori_loop(0, _FROWS, gat_c, 0)
        pltpu.sync_copy(obuf.at[pl.ds(0, _FCH)],
                        eic_out.at[pl.ds(base, _FCH)])

    return k(sk, sv, row, col)


def kernel(x, edge_index, W, b, k):
    row = edge_index[0]
    col = edge_index[1]
    xb32 = x.astype(jnp.bfloat16).astype(jnp.float32)
    er0, er1 = _gather_rows(xb32, row, col)
    k_static = edge_index.shape[1] // 2
    k_residual = (jnp.asarray(k) - k_static).astype(jnp.float32)
    c = (b[0] + k_residual).reshape(1)
    pred = _score(er0, er1, W, c).reshape(-1)
    pred_bits = lax.bitcast_convert_type(pred, jnp.int32)
    sk, sv, _hk, _hv = _sort_topk(pred_bits)
    causal_vals, eir, eic = _finalize(sk, sv, row, col)
    causal_idx = sv
    causal_edge_index = jnp.stack([eir, eic])
    return (causal_vals, causal_idx, causal_edge_index)


# speedup vs baseline: 1.1326x; 1.0415x over previous
import functools

import jax
import jax.numpy as jnp
import numpy as np
from jax import lax
from jax.experimental import pallas as pl
from jax.experimental.pallas import tpu as pltpu
from jax.experimental.pallas import tpu_sc as plsc

_DN = (((1,), (0,)), ((), ()))
_BLK = 2000

_info = plsc.get_sparse_core_info()
_NC, _NS = _info.num_cores, _info.num_subcores
_NW = _NC * _NS

_E = 320000
_K = _E // 2
_NT = 16
_CHUNK = _E // _NT
_SUB = _CHUNK // 16
_R = 256
_NPASS = 4
_SROWS = _CHUNK // 128 + 1
_PAD = 4096
_SIGN = np.int32(-2147483648)

_OCHUNK = _K // _NT


def _dot_body(c_ref, a_ref, b_ref, w_ref, o_ref):
    er = jnp.concatenate([a_ref[...], b_ref[...]], axis=1).astype(jnp.bfloat16)
    acc = lax.dot_general(er, w_ref[...], _DN, preferred_element_type=jnp.float32)
    o_ref[...] = acc + c_ref[0]


def _score(er0, er1, W, c):
    E = er0.shape[0]
    n_blk = E // _BLK
    return pl.pallas_call(
        _dot_body,
        grid=(n_blk,),
        in_specs=[
            pl.BlockSpec(memory_space=pltpu.SMEM),
            pl.BlockSpec((_BLK, 128), lambda i: (i, 0)),
            pl.BlockSpec((_BLK, 128), lambda i: (i, 0)),
            pl.BlockSpec((256, 1), lambda i: (0, 0)),
        ],
        out_specs=pl.BlockSpec((_BLK, 1), lambda i: (i, 0)),
        out_shape=jax.ShapeDtypeStruct((E, 1), jnp.float32),
    )(c, er0, er1, W)


def _gather_rows(xb32, row, col):
    N, D = xb32.shape
    E = row.shape[0]
    per_w = E // _NW
    CH = 400
    n_ch = per_w // CH
    mesh = plsc.VectorSubcoreMesh(core_axis_name="c", subcore_axis_name="s")

    @functools.partial(
        pl.kernel,
        mesh=mesh,
        out_type=[
            jax.ShapeDtypeStruct((E, D), jnp.float32),
            jax.ShapeDtypeStruct((E, D), jnp.float32),
        ],
        scratch_types=[
            pltpu.VMEM((CH,), jnp.int32),
            pltpu.VMEM((CH,), jnp.int32),
            pltpu.VMEM((CH, 128), jnp.float32),
            pltpu.VMEM((CH, 128), jnp.float32),
            pltpu.SemaphoreType.DMA,
            pltpu.SemaphoreType.DMA,
        ],
    )
    def k(x_hbm, row_hbm, col_hbm, er0_hbm, er1_hbm,
          idx_r, idx_c, rows_r, rows_c, sem1, sem2):
        wid = lax.axis_index("s") * _NC + lax.axis_index("c")
        base = wid * per_w

        def body(j, _):
            off = base + j * CH
            pltpu.sync_copy(row_hbm.at[pl.ds(off, CH)], idx_r)
            pltpu.sync_copy(col_hbm.at[pl.ds(off, CH)], idx_c)
            a1 = pltpu.async_copy(x_hbm.at[idx_r], rows_r, sem1)
            a2 = pltpu.async_copy(x_hbm.at[idx_c], rows_c, sem2)
            a1.wait()
            pltpu.sync_copy(rows_r, er0_hbm.at[pl.ds(off, CH), :])
            a2.wait()
            pltpu.sync_copy(rows_c, er1_hbm.at[pl.ds(off, CH), :])
            return 0

        lax.fori_loop(0, n_ch, body, 0)

    return k(xb32, row, col)


def _sort_topk(pred_bits):
    mesh = plsc.VectorSubcoreMesh(
        core_axis_name="c", subcore_axis_name="s", num_cores=1)

    @functools.partial(
        pl.kernel,
        mesh=mesh,
        out_type=[
            jax.ShapeDtypeStruct((_K,), jnp.int32),
            jax.ShapeDtypeStruct((_K,), jnp.int32),
            jax.ShapeDtypeStruct((_E + _PAD,), jnp.int32),
            jax.ShapeDtypeStruct((_E + _PAD,), jnp.int32),
        ],
        scratch_types=[
            pltpu.VMEM_SHARED((_E + _PAD,), jnp.int32),
            pltpu.VMEM_SHARED((_E + _PAD,), jnp.int32),
            pltpu.VMEM_SHARED((_R * _NT,), jnp.int32),
            pltpu.VMEM((_SROWS * 128,), jnp.int32),
            pltpu.VMEM((_SROWS * 128,), jnp.int32),
            pltpu.VMEM((_SROWS, 128), jnp.int32),
            pltpu.VMEM((_R * 16,), jnp.int32),
            pltpu.VMEM((_R * 16,), jnp.int32),
            pltpu.VMEM((_R * _NT,), jnp.int32),
            pltpu.VMEM((_R,), jnp.int32),
            pltpu.VMEM((16,), jnp.int32),
            pltpu.SemaphoreType.DMA,
            pltpu.SemaphoreType.DMA,
        ],
        compiler_params=pltpu.CompilerParams(needs_layout_passes=False),
    )
    def k(pred_hbm, key_out, idx_out, skA, svA,
          skB, svB, hist_sh, ck, cv, dests, lhist, counters,
          grid_v, totals_v, spill_v, sem, sem2):
        cid = lax.axis_index("c")
        sid = lax.axis_index("s")
        t = sid
        lane = jnp.arange(16, dtype=jnp.int32)
        cbase = t * _CHUNK

        @pl.when(cid == 0)
        def _sort():
            def fill_iota(jj, _):
                cv[pl.ds(jj * 16, 16)] = cbase + jj * 16 + lane
                return 0
            lax.fori_loop(0, _CHUNK // 16, fill_iota, 0)

            for p in range(_NPASS):
                src_k, src_v = skA, svA
                dst_k, dst_v = skB, svB
                shift = 8 * p
                if p == 0:
                    pltpu.sync_copy(pred_hbm.at[pl.ds(cbase, _CHUNK)],
                                    ck.at[pl.ds(0, _CHUNK)])

                    def xform(jj, _):
                        bb = ck[pl.ds(jj * 16, 16)]
                        ck[pl.ds(jj * 16, 16)] = jnp.where(
                            bb < 0, bb,
                            jnp.bitwise_xor(jnp.bitwise_not(bb), _SIGN))
                        return 0
                    lax.fori_loop(0, _CHUNK // 16, xform, 0)
                else:
                    pltpu.sync_copy(src_k.at[pl.ds(cbase, _CHUNK)],
                                    ck.at[pl.ds(0, _CHUNK)])
                    pltpu.sync_copy(src_v.at[pl.ds(cbase, _CHUNK)],
                                    cv.at[pl.ds(0, _CHUNK)])

                def zero_h(i, _):
                    lhist[pl.ds(i * 16, 16)] = jnp.zeros((16,), jnp.int32)
                    return 0
                lax.fori_loop(0, _R, zero_h, 0)

                def hist_body(j, _):
                    kk = plsc.load_gather(ck, [lane * _SUB + j])
                    d = lax.shift_right_logical(kk, shift) & 255
                    cidx = lane * _R + d
                    old = plsc.load_gather(lhist, [cidx])
                    plsc.store_scatter(lhist, [cidx], old + 1)
                    return 0
                lax.fori_loop(0, _SUB, hist_body, 0)

                def tot_body(q, _):
                    d = q * 16 + lane
                    acc = jnp.zeros((16,), jnp.int32)
                    for l in range(16):
                        acc = acc + plsc.load_gather(lhist, [l * _R + d])
                    totals_v[pl.ds(q * 16, 16)] = acc
                    return 0
                lax.fori_loop(0, _R // 16, tot_body, 0)
                pltpu.sync_copy(totals_v, hist_sh.at[pl.ds(t * _R, _R)])
                plsc.subcore_barrier()

                pltpu.sync_copy(hist_sh, grid_v)

                def scan_body(d, run):
                    v = plsc.load_gather(grid_v, [lane * _R + d])
                    cs = plsc.cumsum(v)
                    ex = cs - v
                    spill_v[...] = run + ex
                    my_base = plsc.load_gather(
                        spill_v, [jnp.full((16,), t, jnp.int32)])
                    lh = plsc.load_gather(lhist, [lane * _R + d])
                    lcs = plsc.cumsum(lh)
                    lex = lcs - lh
                    plsc.store_scatter(counters, [lane * _R + d], my_base + lex)
                    spill_v[...] = cs
                    tot = plsc.load_gather(
                        spill_v, [jnp.full((16,), 15, jnp.int32)])
                    return run + tot

                lax.fori_loop(0, _R, scan_body, jnp.zeros((16,), jnp.int32))

                def rank_body(j, _):
                    pos = lane * _SUB + j
                    kk = plsc.load_gather(ck, [pos])
                    d = lax.shift_right_logical(kk, shift) & 255
                    cidx = lane * _R + d
                    dest = plsc.load_gather(counters, [cidx])
                    plsc.store_scatter(counters, [cidx], dest + 1)
                    plsc.store_scatter(dests, [pos // 128, pos % 128], dest)
                    return 0
                lax.fori_loop(0, _SUB, rank_body, 0)

                def pad_body(j, _):
                    pos = _CHUNK + j * 16 + lane
                    plsc.store_scatter(dests, [pos // 128, pos % 128],
                                       _E + t * 128 + j * 16 + lane)
                    return 0
                lax.fori_loop(0, (_SROWS * 128 - _CHUNK) // 16, pad_body, 0)

                def scat_body(f, _):
                    a1 = pltpu.async_copy(ck.at[pl.ds(f * 128, 128)],
                                          dst_k.at[dests.at[f]], sem)
                    a2 = pltpu.async_copy(cv.at[pl.ds(f * 128, 128)],
                                          dst_v.at[dests.at[f]], sem2)
                    a1.wait()
                    a2.wait()
                    return 0
                lax.fori_loop(0, _SROWS, scat_body, 0)
                plsc.subcore_barrier()

                if p < _NPASS - 1:
                    pltpu.sync_copy(skB.at[pl.ds(cbase, _CHUNK)],
                                    ck.at[pl.ds(0, _CHUNK)])
                    pltpu.sync_copy(ck.at[pl.ds(0, _CHUNK)],
                                    skA.at[pl.ds(cbase, _CHUNK)])
                    pltpu.sync_copy(svB.at[pl.ds(cbase, _CHUNK)],
                                    cv.at[pl.ds(0, _CHUNK)])
                    pltpu.sync_copy(cv.at[pl.ds(0, _CHUNK)],
                                    svA.at[pl.ds(cbase, _CHUNK)])
                    plsc.subcore_barrier()

            fin_k, fin_v = skB, svB
            obase = t * _OCHUNK
            pltpu.sync_copy(fin_k.at[pl.ds(obase, _OCHUNK)],
                            ck.at[pl.ds(0, _OCHUNK)])
            pltpu.sync_copy(fin_v.at[pl.ds(obase, _OCHUNK)],
                            cv.at[pl.ds(0, _OCHUNK)])
            pltpu.sync_copy(ck.at[pl.ds(0, _OCHUNK)],
                            key_out.at[pl.ds(obase, _OCHUNK)])
            pltpu.sync_copy(cv.at[pl.ds(0, _OCHUNK)],
                            idx_out.at[pl.ds(obase, _OCHUNK)])

    return k(pred_bits)


_FCH = _K // _NW
_FROWS = _FCH // 128 + 1


def _finalize(sk, sv, row, col):
    mesh = plsc.VectorSubcoreMesh(core_axis_name="c", subcore_axis_name="s")

    @functools.partial(
        pl.kernel,
        mesh=mesh,
        out_type=[
            jax.ShapeDtypeStruct((_K,), jnp.float32),
            jax.ShapeDtypeStruct((_K,), jnp.int32),
            jax.ShapeDtypeStruct((_K,), jnp.int32),
        ],
        scratch_types=[
            pltpu.VMEM((_FCH,), jnp.int32),
            pltpu.VMEM((_FCH,), jnp.float32),
            pltpu.VMEM((_FROWS, 128), jnp.int32),
            pltpu.VMEM((_FROWS * 128,), jnp.int32),
            pltpu.SemaphoreType.DMA,
        ],
        compiler_params=pltpu.CompilerParams(needs_layout_passes=False),
    )
    def k(sk_hbm, sv_hbm, row_hbm, col_hbm, vals_out, eir_out, eic_out,
          buf, fvals, oidx, obuf, sem):
        wid = lax.axis_index("s") * _NC + lax.axis_index("c")
        base = wid * _FCH
        lane = jnp.arange(16, dtype=jnp.int32)

        pltpu.sync_copy(sk_hbm.at[pl.ds(base, _FCH)], buf)

        def inv_one(off):
            k2 = buf[pl.ds(off, 16)]
            bb = jnp.where(k2 < 0, k2,
                           jnp.bitwise_xor(jnp.bitwise_not(k2), _SIGN))
            fvals[pl.ds(off, 16)] = plsc.bitcast(bb, jnp.float32)

        def inv_body(jj, _):
            inv_one(jj * 16)
            return 0
        lax.fori_loop(0, _FCH // 16, inv_body, 0)
        if _FCH % 16:
            inv_one(_FCH - 16)
        pltpu.sync_copy(fvals, vals_out.at[pl.ds(base, _FCH)])

        pltpu.sync_copy(sv_hbm.at[pl.ds(base, _FCH)], buf)

        def oidx_fill(jj, _):
            pos = jj * 16 + lane
            v = jnp.where(pos < _FCH,
                          plsc.load_gather(buf, [jnp.minimum(pos, _FCH - 1)]),
                          0)
            plsc.store_scatter(oidx, [pos // 128, pos % 128], v)
            return 0
        lax.fori_loop(0, (_FROWS * 128) // 16, oidx_fill, 0)

        def gat_r(f, _):
            pltpu.async_copy(row_hbm.at[oidx.at[f]],
                             obuf.at[pl.ds(f * 128, 128)], sem)
            pltpu.make_async_copy(row_hbm.at[oidx.at[f]],
                                  obuf.at[pl.ds(f * 128, 128)], sem).wait()
            return 0
        lax.fori_loop(0, _FROWS, gat_r, 0)
        pltpu.sync_copy(obuf.at[pl.ds(0, _FCH)],
                        eir_out.at[pl.ds(base, _FCH)])

        def gat_c(f, _):
            pltpu.async_copy(col_hbm.at[oidx.at[f]],
                             obuf.at[pl.ds(f * 128, 128)], sem)
            pltpu.make_async_copy(col_hbm.at[oidx.at[f]],
                                  obuf.at[pl.ds(f * 128, 128)], sem).wait()
            return 0
        lax.fori_loop(0, _FROWS, gat_c, 0)
        pltpu.sync_copy(obuf.at[pl.ds(0, _FCH)],
                        eic_out.at[pl.ds(base, _FCH)])

    return k(sk, sv, row, col)


def kernel(x, edge_index, W, b, k):
    row = edge_index[0]
    col = edge_index[1]
    xb32 = x.astype(jnp.bfloat16).astype(jnp.float32)
    er0, er1 = _gather_rows(xb32, row, col)
    k_static = edge_index.shape[1] // 2
    k_residual = (jnp.asarray(k) - k_static).astype(jnp.float32)
    c = (b[0] + k_residual).reshape(1)
    pred = _score(er0, er1, W, c).reshape(-1)
    pred_bits = lax.bitcast_convert_type(pred, jnp.int32)
    sk, sv, _hk, _hv = _sort_topk(pred_bits)
    causal_vals, eir, eic = _finalize(sk, sv, row, col)
    causal_idx = sv
    causal_edge_index = jnp.stack([eir, eic])
    return (causal_vals, causal_idx, causal_edge_index)


# speedup vs baseline: 1.1790x; 1.0409x over previous
import functools

import jax
import jax.numpy as jnp
import numpy as np
from jax import lax
from jax.experimental import pallas as pl
from jax.experimental.pallas import tpu as pltpu
from jax.experimental.pallas import tpu_sc as plsc

_DN = (((1,), (0,)), ((), ()))
_BLK = 2000

_info = plsc.get_sparse_core_info()
_NC, _NS = _info.num_cores, _info.num_subcores
_NW = _NC * _NS

_E = 320000
_K = _E // 2
_NT = 16
_CHUNK = _E // _NT
_SUB = _CHUNK // 16
_R = 256
_NPASS = 4
_SROWS = _CHUNK // 128 + 1
_PAD = 4096
_SIGN = np.int32(-2147483648)

_OCHUNK = _K // _NT


def _dot_body(c_ref, a_ref, b_ref, w_ref, o_ref):
    er = jnp.concatenate([a_ref[...], b_ref[...]], axis=1).astype(jnp.bfloat16)
    acc = lax.dot_general(er, w_ref[...], _DN, preferred_element_type=jnp.float32)
    o_ref[...] = acc + c_ref[0]


def _score(er0, er1, W, c):
    E = er0.shape[0]
    n_blk = E // _BLK
    return pl.pallas_call(
        _dot_body,
        grid=(n_blk,),
        in_specs=[
            pl.BlockSpec(memory_space=pltpu.SMEM),
            pl.BlockSpec((_BLK, 128), lambda i: (i, 0)),
            pl.BlockSpec((_BLK, 128), lambda i: (i, 0)),
            pl.BlockSpec((256, 1), lambda i: (0, 0)),
        ],
        out_specs=pl.BlockSpec((_BLK, 1), lambda i: (i, 0)),
        out_shape=jax.ShapeDtypeStruct((E, 1), jnp.float32),
    )(c, er0, er1, W)


def _gather_rows(xb32, row, col):
    N, D = xb32.shape
    E = row.shape[0]
    per_w = E // _NW
    CH = 400
    n_ch = per_w // CH
    mesh = plsc.VectorSubcoreMesh(core_axis_name="c", subcore_axis_name="s")

    @functools.partial(
        pl.kernel,
        mesh=mesh,
        out_type=[
            jax.ShapeDtypeStruct((E, D), jnp.float32),
            jax.ShapeDtypeStruct((E, D), jnp.float32),
        ],
        scratch_types=[
            pltpu.VMEM((CH,), jnp.int32),
            pltpu.VMEM((CH,), jnp.int32),
            pltpu.VMEM((CH, 128), jnp.float32),
            pltpu.VMEM((CH, 128), jnp.float32),
            pltpu.SemaphoreType.DMA,
            pltpu.SemaphoreType.DMA,
        ],
    )
    def k(x_hbm, row_hbm, col_hbm, er0_hbm, er1_hbm,
          idx_r, idx_c, rows_r, rows_c, sem1, sem2):
        wid = lax.axis_index("s") * _NC + lax.axis_index("c")
        base = wid * per_w

        def body(j, _):
            off = base + j * CH
            pltpu.sync_copy(row_hbm.at[pl.ds(off, CH)], idx_r)
            pltpu.sync_copy(col_hbm.at[pl.ds(off, CH)], idx_c)
            a1 = pltpu.async_copy(x_hbm.at[idx_r], rows_r, sem1)
            a2 = pltpu.async_copy(x_hbm.at[idx_c], rows_c, sem2)
            a1.wait()
            pltpu.sync_copy(rows_r, er0_hbm.at[pl.ds(off, CH), :])
            a2.wait()
            pltpu.sync_copy(rows_c, er1_hbm.at[pl.ds(off, CH), :])
            return 0

        lax.fori_loop(0, n_ch, body, 0)

    return k(xb32, row, col)


def _sort_topk(pred_bits):
    mesh = plsc.VectorSubcoreMesh(
        core_axis_name="c", subcore_axis_name="s", num_cores=1)

    @functools.partial(
        pl.kernel,
        mesh=mesh,
        out_type=[
            jax.ShapeDtypeStruct((_K,), jnp.int32),
            jax.ShapeDtypeStruct((_K,), jnp.int32),
            jax.ShapeDtypeStruct((_E + _PAD,), jnp.int32),
            jax.ShapeDtypeStruct((_E + _PAD,), jnp.int32),
        ],
        scratch_types=[
            pltpu.VMEM_SHARED((_E + _PAD,), jnp.int32),
            pltpu.VMEM_SHARED((_E + _PAD,), jnp.int32),
            pltpu.VMEM_SHARED((_R * _NT,), jnp.int32),
            pltpu.VMEM((_SROWS * 128,), jnp.int32),
            pltpu.VMEM((_SROWS * 128,), jnp.int32),
            pltpu.VMEM((_SROWS, 128), jnp.int32),
            pltpu.VMEM((_R * 16,), jnp.int32),
            pltpu.VMEM((_R * 16,), jnp.int32),
            pltpu.VMEM((_R * _NT,), jnp.int32),
            pltpu.VMEM((_R,), jnp.int32),
            pltpu.VMEM((16,), jnp.int32),
            pltpu.SemaphoreType.DMA,
            pltpu.SemaphoreType.DMA,
        ],
        compiler_params=pltpu.CompilerParams(needs_layout_passes=False),
    )
    def k(pred_hbm, key_out, idx_out, skA, svA,
          skB, svB, hist_sh, ck, cv, dests, lhist, counters,
          grid_v, totals_v, spill_v, sem, sem2):
        cid = lax.axis_index("c")
        sid = lax.axis_index("s")
        t = sid
        lane = jnp.arange(16, dtype=jnp.int32)
        cbase = t * _CHUNK

        @pl.when(cid == 0)
        def _sort():
            def fill_iota(jj, _):
                cv[pl.ds(jj * 16, 16)] = cbase + jj * 16 + lane
                return 0
            lax.fori_loop(0, _CHUNK // 16, fill_iota, 0)

            for p in range(_NPASS):
                src_k, src_v = skA, svA
                dst_k, dst_v = skB, svB
                shift = 8 * p
                if p == 0:
                    pltpu.sync_copy(pred_hbm.at[pl.ds(cbase, _CHUNK)],
                                    ck.at[pl.ds(0, _CHUNK)])

                    def xform(jj, _):
                        bb = ck[pl.ds(jj * 16, 16)]
                        ck[pl.ds(jj * 16, 16)] = jnp.where(
                            bb < 0, bb,
                            jnp.bitwise_xor(jnp.bitwise_not(bb), _SIGN))
                        return 0
                    lax.fori_loop(0, _CHUNK // 16, xform, 0)
                else:
                    pltpu.sync_copy(src_k.at[pl.ds(cbase, _CHUNK)],
                                    ck.at[pl.ds(0, _CHUNK)])
                    pltpu.sync_copy(src_v.at[pl.ds(cbase, _CHUNK)],
                                    cv.at[pl.ds(0, _CHUNK)])

                def zero_h(i, _):
                    lhist[pl.ds(i * 16, 16)] = jnp.zeros((16,), jnp.int32)
                    return 0
                lax.fori_loop(0, _R, zero_h, 0)

                def hist_body(j, _):
                    kk = plsc.load_gather(ck, [lane * _SUB + j])
                    d = lax.shift_right_logical(kk, shift) & 255
                    cidx = lane * _R + d
                    old = plsc.load_gather(lhist, [cidx])
                    plsc.store_scatter(lhist, [cidx], old + 1)
                    return 0
                lax.fori_loop(0, _SUB, hist_body, 0)

                def tot_body(q, _):
                    d = q * 16 + lane
                    acc = jnp.zeros((16,), jnp.int32)
                    for l in range(16):
                        acc = acc + plsc.load_gather(lhist, [l * _R + d])
                    totals_v[pl.ds(q * 16, 16)] = acc
                    return 0
                lax.fori_loop(0, _R // 16, tot_body, 0)
                pltpu.sync_copy(totals_v, hist_sh.at[pl.ds(t * _R, _R)])
                plsc.subcore_barrier()

                pltpu.sync_copy(hist_sh, grid_v)

                def scan_body(d, run):
                    v = plsc.load_gather(grid_v, [lane * _R + d])
                    cs = plsc.cumsum(v)
                    ex = cs - v
                    spill_v[...] = run + ex
                    my_base = plsc.load_gather(
                        spill_v, [jnp.full((16,), t, jnp.int32)])
                    lh = plsc.load_gather(lhist, [lane * _R + d])
                    lcs = plsc.cumsum(lh)
                    lex = lcs - lh
                    plsc.store_scatter(counters, [lane * _R + d], my_base + lex)
                    spill_v[...] = cs
                    tot = plsc.load_gather(
                        spill_v, [jnp.full((16,), 15, jnp.int32)])
                    return run + tot

                lax.fori_loop(0, _R, scan_body, jnp.zeros((16,), jnp.int32))

                def rank_body(j, _):
                    pos = lane * _SUB + j
                    kk = plsc.load_gather(ck, [pos])
                    d = lax.shift_right_logical(kk, shift) & 255
                    cidx = lane * _R + d
                    dest = plsc.load_gather(counters, [cidx])
                    plsc.store_scatter(counters, [cidx], dest + 1)
                    plsc.store_scatter(dests, [pos // 128, pos % 128], dest)
                    return 0
                lax.fori_loop(0, _SUB, rank_body, 0)

                def pad_body(j, _):
                    pos = _CHUNK + j * 16 + lane
                    plsc.store_scatter(dests, [pos // 128, pos % 128],
                                       _E + t * 128 + j * 16 + lane)
                    return 0
                lax.fori_loop(0, (_SROWS * 128 - _CHUNK) // 16, pad_body, 0)

                def scat_body(f, _):
                    a1 = pltpu.async_copy(ck.at[pl.ds(f * 128, 128)],
                                          dst_k.at[dests.at[f]], sem)
                    a2 = pltpu.async_copy(cv.at[pl.ds(f * 128, 128)],
                                          dst_v.at[dests.at[f]], sem2)
                    a1.wait()
                    a2.wait()
                    return 0
                lax.fori_loop(0, _SROWS, scat_body, 0)
                plsc.subcore_barrier()

                if p < _NPASS - 1:
                    pltpu.sync_copy(skB.at[pl.ds(cbase, _CHUNK)],
                                    ck.at[pl.ds(0, _CHUNK)])
                    pltpu.sync_copy(ck.at[pl.ds(0, _CHUNK)],
                                    skA.at[pl.ds(cbase, _CHUNK)])
                    pltpu.sync_copy(svB.at[pl.ds(cbase, _CHUNK)],
                                    cv.at[pl.ds(0, _CHUNK)])
                    pltpu.sync_copy(cv.at[pl.ds(0, _CHUNK)],
                                    svA.at[pl.ds(cbase, _CHUNK)])
                    plsc.subcore_barrier()

            fin_k, fin_v = skB, svB
            obase = t * _OCHUNK
            pltpu.sync_copy(fin_k.at[pl.ds(obase, _OCHUNK)],
                            ck.at[pl.ds(0, _OCHUNK)])
            pltpu.sync_copy(fin_v.at[pl.ds(obase, _OCHUNK)],
                            cv.at[pl.ds(0, _OCHUNK)])
            pltpu.sync_copy(ck.at[pl.ds(0, _OCHUNK)],
                            key_out.at[pl.ds(obase, _OCHUNK)])
            pltpu.sync_copy(cv.at[pl.ds(0, _OCHUNK)],
                            idx_out.at[pl.ds(obase, _OCHUNK)])

    return k(pred_bits)


_FCH = _K // _NW
_FROWS = _FCH // 128 + 1


def _finalize(sk, sv, row, col):
    mesh = plsc.VectorSubcoreMesh(core_axis_name="c", subcore_axis_name="s")

    @functools.partial(
        pl.kernel,
        mesh=mesh,
        out_type=[
            jax.ShapeDtypeStruct((_K,), jnp.float32),
            jax.ShapeDtypeStruct((_K,), jnp.int32),
            jax.ShapeDtypeStruct((_K,), jnp.int32),
        ],
        scratch_types=[
            pltpu.VMEM((_FCH,), jnp.int32),
            pltpu.VMEM((_FCH,), jnp.float32),
            pltpu.VMEM((_FROWS, 128), jnp.int32),
            pltpu.VMEM((_FROWS * 128,), jnp.int32),
            pltpu.VMEM((_FROWS * 128,), jnp.int32),
            pltpu.SemaphoreType.DMA,
            pltpu.SemaphoreType.DMA,
        ],
        compiler_params=pltpu.CompilerParams(needs_layout_passes=False),
    )
    def k(sk_hbm, sv_hbm, row_hbm, col_hbm, vals_out, eir_out, eic_out,
          buf, fvals, oidx, obuf, obuf2, sem, sem2):
        wid = lax.axis_index("s") * _NC + lax.axis_index("c")
        base = wid * _FCH
        lane = jnp.arange(16, dtype=jnp.int32)

        pltpu.sync_copy(sk_hbm.at[pl.ds(base, _FCH)], buf)

        def inv_one(off):
            k2 = buf[pl.ds(off, 16)]
            bb = jnp.where(k2 < 0, k2,
                           jnp.bitwise_xor(jnp.bitwise_not(k2), _SIGN))
            fvals[pl.ds(off, 16)] = plsc.bitcast(bb, jnp.float32)

        def inv_body(jj, _):
            inv_one(jj * 16)
            return 0
        lax.fori_loop(0, _FCH // 16, inv_body, 0)
        if _FCH % 16:
            inv_one(_FCH - 16)
        pltpu.sync_copy(fvals, vals_out.at[pl.ds(base, _FCH)])

        pltpu.sync_copy(sv_hbm.at[pl.ds(base, _FCH)], buf)

        def oidx_fill(jj, _):
            pos = jj * 16 + lane
            v = jnp.where(pos < _FCH,
                          plsc.load_gather(buf, [jnp.minimum(pos, _FCH - 1)]),
                          0)
            plsc.store_scatter(oidx, [pos // 128, pos % 128], v)
            return 0
        lax.fori_loop(0, (_FROWS * 128) // 16, oidx_fill, 0)

        def gat_rc(f, _):
            a1 = pltpu.async_copy(row_hbm.at[oidx.at[f]],
                                  obuf.at[pl.ds(f * 128, 128)], sem)
            a2 = pltpu.async_copy(col_hbm.at[oidx.at[f]],
                                  obuf2.at[pl.ds(f * 128, 128)], sem2)
            a1.wait()
            a2.wait()
            return 0
        lax.fori_loop(0, _FROWS, gat_rc, 0)
        pltpu.sync_copy(obuf.at[pl.ds(0, _FCH)],
                        eir_out.at[pl.ds(base, _FCH)])
        pltpu.sync_copy(obuf2.at[pl.ds(0, _FCH)],
                        eic_out.at[pl.ds(base, _FCH)])

    return k(sk, sv, row, col)


def kernel(x, edge_index, W, b, k):
    row = edge_index[0]
    col = edge_index[1]
    xb32 = x.astype(jnp.bfloat16).astype(jnp.float32)
    er0, er1 = _gather_rows(xb32, row, col)
    k_static = edge_index.shape[1] // 2
    k_residual = (jnp.asarray(k) - k_static).astype(jnp.float32)
    c = (b[0] + k_residual).reshape(1)
    pred = _score(er0, er1, W, c).reshape(-1)
    pred_bits = lax.bitcast_convert_type(pred, jnp.int32)
    sk, sv, _hk, _hv = _sort_topk(pred_bits)
    causal_vals, eir, eic = _finalize(sk, sv, row, col)
    causal_idx = sv
    causal_edge_index = jnp.stack([eir, eic])
    return (causal_vals, causal_idx, causal_edge_index)
